# baseline (speedup 1.0000x reference)
import jax
import jax.numpy as jnp
from jax.experimental import pallas as pl
from jax.experimental.pallas import tpu as pltpu

_K = 64
_ROWS = 4096
_CBLK = 256
_NGRP = 8


def _oddeven_comparators(n):
    result = []

    def merge(lo, n2, r):
        m = r * 2
        if m < n2:
            merge(lo, n2, m)
            merge(lo + r, n2, m)
            for i in range(lo + r, lo + n2 - r, m):
                result.append((i, i + r))
        else:
            result.append((lo, lo + r))

    def sort(lo, n2):
        if n2 > 1:
            m = n2 // 2
            sort(lo, m)
            sort(lo + m, m)
            merge(lo, n2, 1)

    sort(0, n)
    return result


_SORT_NET = _oddeven_comparators(_K)


def _oddeven_sort_desc(tiles):
    for i, p in _SORT_NET:
        a, b = tiles[i], tiles[p]
        tiles[i], tiles[p] = jnp.maximum(a, b), jnp.minimum(a, b)
    return tiles


def _bitonic_merge_desc(tiles):
    n = len(tiles)
    d = n // 2
    while d >= 1:
        for i in range(n):
            p = i ^ d
            if p > i:
                a, b = tiles[i], tiles[p]
                tiles[i], tiles[p] = jnp.maximum(a, b), jnp.minimum(a, b)
        d //= 2
    return tiles


def _merge_groups(ga, gb):
    nt = [jnp.maximum(ga[i], gb[_K - 1 - i]) for i in range(_K)]
    return _bitonic_merge_desc(nt)


def _topk_kernel(x_ref, o_ref):
    groups = []
    for g in range(_NGRP):
        tiles = [x_ref[0, pl.ds(_K * i + 8 * g, 8), :] for i in range(_K)]
        groups.append(_oddeven_sort_desc(tiles))
    while len(groups) > 1:
        half = len(groups) // 2
        groups = [_merge_groups(groups[g], groups[g + half]) for g in range(half)]
    tiles = groups[0]
    width = 8
    while width > 1:
        half = width // 2
        nt = [
            jnp.maximum(tiles[i][:half, :], tiles[_K - 1 - i][half:width, :])
            for i in range(_K)
        ]
        tiles = _bitonic_merge_desc(nt)
        width = half
    o_ref[0, :, :] = jnp.concatenate(tiles, axis=0)


def kernel(x):
    b, rows, cols = x.shape
    grid = (b, cols // _CBLK)
    return pl.pallas_call(
        _topk_kernel,
        grid=grid,
        in_specs=[pl.BlockSpec((1, rows, _CBLK), lambda i, j: (i, 0, j))],
        out_specs=pl.BlockSpec((1, _K, _CBLK), lambda i, j: (i, 0, j)),
        out_shape=jax.ShapeDtypeStruct((b, _K, cols), x.dtype),
        compiler_params=pltpu.CompilerParams(
            dimension_semantics=("parallel", "parallel"),
        ),
    )(x)


# speedup vs baseline: 1.0087x; 1.0087x over previous
import jax
import jax.numpy as jnp
from jax.experimental import pallas as pl
from jax.experimental.pallas import tpu as pltpu

_K = 64
_ROWS = 4096
_CBLK = 128
_NGRP = 8


def _oddeven_comparators(n):
    result = []

    def merge(lo, n2, r):
        m = r * 2
        if m < n2:
            merge(lo, n2, m)
            merge(lo + r, n2, m)
            for i in range(lo + r, lo + n2 - r, m):
                result.append((i, i + r))
        else:
            result.append((lo, lo + r))

    def sort(lo, n2):
        if n2 > 1:
            m = n2 // 2
            sort(lo, m)
            sort(lo + m, m)
            merge(lo, n2, 1)

    sort(0, n)
    return result


_SORT_NET = _oddeven_comparators(_K)


def _oddeven_sort_desc(tiles):
    for i, p in _SORT_NET:
        a, b = tiles[i], tiles[p]
        tiles[i], tiles[p] = jnp.maximum(a, b), jnp.minimum(a, b)
    return tiles


def _bitonic_merge_desc(tiles):
    n = len(tiles)
    d = n // 2
    while d >= 1:
        for i in range(n):
            p = i ^ d
            if p > i:
                a, b = tiles[i], tiles[p]
                tiles[i], tiles[p] = jnp.maximum(a, b), jnp.minimum(a, b)
        d //= 2
    return tiles


def _merge_groups(ga, gb):
    nt = [jnp.maximum(ga[i], gb[_K - 1 - i]) for i in range(_K)]
    return _bitonic_merge_desc(nt)


def _topk_kernel(x_ref, o_ref):
    groups = []
    for g in range(_NGRP):
        tiles = [x_ref[0, pl.ds(_K * i + 8 * g, 8), :] for i in range(_K)]
        groups.append(_oddeven_sort_desc(tiles))
    while len(groups) > 1:
        half = len(groups) // 2
        groups = [_merge_groups(groups[g], groups[g + half]) for g in range(half)]
    tiles = groups[0]
    width = 8
    while width > 1:
        half = width // 2
        nt = [
            jnp.maximum(tiles[i][:half, :], tiles[_K - 1 - i][half:width, :])
            for i in range(_K)
        ]
        tiles = _bitonic_merge_desc(nt)
        width = half
    o_ref[0, :, :] = jnp.concatenate(tiles, axis=0)


def kernel(x):
    b, rows, cols = x.shape
    grid = (b, cols // _CBLK)
    return pl.pallas_call(
        _topk_kernel,
        grid=grid,
        in_specs=[pl.BlockSpec((1, rows, _CBLK), lambda i, j: (i, 0, j))],
        out_specs=pl.BlockSpec((1, _K, _CBLK), lambda i, j: (i, 0, j)),
        out_shape=jax.ShapeDtypeStruct((b, _K, cols), x.dtype),
        compiler_params=pltpu.CompilerParams(
            dimension_semantics=("parallel", "parallel"),
        ),
    )(x)


# speedup vs baseline: 1.0147x; 1.0059x over previous
import jax
import jax.numpy as jnp
from jax.experimental import pallas as pl
from jax.experimental.pallas import tpu as pltpu

_K = 64
_ROWS = 4096
_CBLK = 128
_NGRP = 8


def _oddeven_comparators(n):
    result = []

    def merge(lo, n2, r):
        m = r * 2
        if m < n2:
            merge(lo, n2, m)
            merge(lo + r, n2, m)
            for i in range(lo + r, lo + n2 - r, m):
                result.append((i, i + r))
        else:
            result.append((lo, lo + r))

    def sort(lo, n2):
        if n2 > 1:
            m = n2 // 2
            sort(lo, m)
            sort(lo + m, m)
            merge(lo, n2, 1)

    sort(0, n)
    return result


_SORT_NET = _oddeven_comparators(_K)


def _oddeven_sort_desc(tiles):
    for i, p in _SORT_NET:
        a, b = tiles[i], tiles[p]
        tiles[i], tiles[p] = jnp.maximum(a, b), jnp.minimum(a, b)
    return tiles


def _bitonic_merge_desc(tiles):
    n = len(tiles)
    d = n // 2
    while d >= 1:
        for i in range(n):
            p = i ^ d
            if p > i:
                a, b = tiles[i], tiles[p]
                tiles[i], tiles[p] = jnp.maximum(a, b), jnp.minimum(a, b)
        d //= 2
    return tiles


def _merge_groups(ga, gb):
    nt = [jnp.maximum(ga[i], gb[_K - 1 - i]) for i in range(_K)]
    return _bitonic_merge_desc(nt)


def _topk_kernel(x_ref, o_ref):
    groups = []
    for g in range(_NGRP // 2):
        ta = [x_ref[0, pl.ds(_K * i + 8 * g, 8), :] for i in range(_K)]
        tb = [x_ref[0, pl.ds(_K * i + 8 * (g + 4), 8), :] for i in range(_K)]
        ta = _oddeven_sort_desc(ta)
        tb = _oddeven_sort_desc(tb)
        groups.append(_merge_groups(ta, tb))
    while len(groups) > 1:
        half = len(groups) // 2
        groups = [_merge_groups(groups[g], groups[g + half]) for g in range(half)]
    tiles = groups[0]
    width = 8
    while width > 1:
        half = width // 2
        nt = [
            jnp.maximum(tiles[i][:half, :], tiles[_K - 1 - i][half:width, :])
            for i in range(_K)
        ]
        tiles = _bitonic_merge_desc(nt)
        width = half
    o_ref[0, :, :] = jnp.concatenate(tiles, axis=0)


def kernel(x):
    b, rows, cols = x.shape
    grid = (b, cols // _CBLK)
    return pl.pallas_call(
        _topk_kernel,
        grid=grid,
        in_specs=[pl.BlockSpec((1, rows, _CBLK), lambda i, j: (i, 0, j))],
        out_specs=pl.BlockSpec((1, _K, _CBLK), lambda i, j: (i, 0, j)),
        out_shape=jax.ShapeDtypeStruct((b, _K, cols), x.dtype),
        compiler_params=pltpu.CompilerParams(
            dimension_semantics=("parallel", "parallel"),
        ),
    )(x)
